# baseline (speedup 1.0000x reference)
import functools

import jax
import jax.numpy as jnp
from jax import lax
from jax.experimental import pallas as pl
from jax.experimental.pallas import tpu as pltpu
from jax.experimental.pallas import tpu_sc as plsc

_NUM_GRAPHS = 256
_HID = 128
_CHUNK = 128
_SUB = 3
_SCHUNK = _SUB * _CHUNK
_ACC_ROWS = 264
_DUMMY = 256
_CNT_LEN = 384


def _mask_tail_idx(idx_v, thresh):
    if thresh <= 0:
        return
    for k in range(_CHUNK // 16):
        if k * 16 >= thresh:
            break
        vals = idx_v[pl.ds(k * 16, 16)]
        pos = lax.iota(jnp.int32, 16) + (k * 16)
        idx_v[pl.ds(k * 16, 16)] = jnp.where(pos < thresh, _DUMMY, vals)


def _process_array(n_rows, x_hbm, ids_hbm, acc_sh, cnt_sh, idx2_v, rows2_v,
                   ones_v, sems, wid, n_workers):
    n_chunks = (n_rows + _SCHUNK - 1) // _SCHUNK
    assert n_chunks >= 2 * n_workers
    tail_rem = n_rows - (n_chunks - 1) * _SCHUNK
    tail_thresh = _SCHUNK - tail_rem
    n_iters = (n_chunks - wid + n_workers - 1) // n_workers

    def rowbase(t):
        cid = wid + t * n_workers
        is_tail = cid == (n_chunks - 1)
        return jnp.where(is_tail, n_rows - _SCHUNK, cid * _SCHUNK), is_tail

    def start(t, b):
        rb, _ = rowbase(t)
        for k in range(_SUB):
            pltpu.async_copy(ids_hbm.at[pl.ds(rb + k * _CHUNK, _CHUNK)],
                             idx2_v.at[b, k], sems.at[b])
        pltpu.async_copy(x_hbm.at[pl.ds(rb, _SCHUNK)], rows2_v.at[b],
                         sems.at[b])

    def drain(b):
        for k in range(_SUB):
            pltpu.make_async_copy(ids_hbm.at[pl.ds(0, _CHUNK)],
                                  idx2_v.at[b, k], sems.at[b]).wait()
        pltpu.make_async_copy(x_hbm.at[pl.ds(0, _SCHUNK)], rows2_v.at[b],
                              sems.at[b]).wait()

    start(0, 0)
    start(1, 1)

    def pair_body(p, _):
        for b in range(2):
            t = 2 * p + b

            @pl.when(t < n_iters)
            def _():
                drain(b)
                _, is_tail = rowbase(t)
                if tail_thresh > 0:
                    @pl.when(is_tail)
                    def _():
                        for k in range(_SUB):
                            _mask_tail_idx(idx2_v.at[b, k],
                                           tail_thresh - k * _CHUNK)
                for k in range(_SUB):
                    pltpu.sync_copy(rows2_v.at[b, pl.ds(k * _CHUNK, _CHUNK)],
                                    acc_sh.at[idx2_v.at[b, k]], add=True)
                    pltpu.sync_copy(ones_v, cnt_sh.at[idx2_v.at[b, k]],
                                    add=True)

                @pl.when(t + 2 < n_iters)
                def _():
                    start(t + 2, b)
        return 0

    lax.fori_loop(0, (n_iters + 1) // 2, pair_body, 0)


def _segment_sums(x0, b0, x1, b1, zacc):
    info = plsc.get_sparse_core_info()
    nc, ns = info.num_cores, info.num_subcores
    nw = nc * ns
    mesh = plsc.VectorSubcoreMesh(core_axis_name="c", subcore_axis_name="s")
    n0 = x0.shape[0]
    n1 = x1.shape[0]
    rows_per_tile = _NUM_GRAPHS // ns

    @functools.partial(
        pl.kernel,
        out_type=[
            jax.ShapeDtypeStruct((nc, 2, _ACC_ROWS, _HID), jnp.float32),
            jax.ShapeDtypeStruct((nc * 2 * _CNT_LEN,), jnp.float32),
        ],
        mesh=mesh,
        scratch_types=[
            pltpu.VMEM((2, _SUB, _CHUNK), jnp.int32),
            pltpu.VMEM((2, _SCHUNK, _HID), jnp.float32),
            pltpu.VMEM((_CHUNK,), jnp.float32),
            pltpu.VMEM((_CNT_LEN,), jnp.float32),
            pltpu.VMEM_SHARED((2, _ACC_ROWS, _HID), jnp.float32),
            pltpu.VMEM_SHARED((_CNT_LEN,), jnp.float32),
            pltpu.VMEM_SHARED((_CNT_LEN,), jnp.float32),
            pltpu.SemaphoreType.DMA((2,)),
        ],
    )
    def seg_kernel(x0_hbm, b0_hbm, x1_hbm, b1_hbm, zacc_hbm,
                   acc_out, cnt_out, idx2_v, rows2_v, ones_v, cntz_v,
                   acc_sh, cnt0_sh, cnt1_sh, sems):
        c = lax.axis_index("c")
        s = lax.axis_index("s")
        wid = s * nc + c

        one = jnp.full((16,), 1.0, jnp.float32)
        for i in range(_CHUNK // 16):
            ones_v[pl.ds(i * 16, 16)] = one

        @pl.when(s == 0)
        def _():
            pltpu.sync_copy(zacc_hbm, acc_sh.at[0])
            pltpu.sync_copy(zacc_hbm, acc_sh.at[1])
            zero = jnp.zeros((16,), jnp.float32)
            for i in range(_CNT_LEN // 16):
                cntz_v[pl.ds(i * 16, 16)] = zero
            pltpu.sync_copy(cntz_v, cnt0_sh)
            pltpu.sync_copy(cntz_v, cnt1_sh)

        plsc.subcore_barrier()

        _process_array(n0, x0_hbm, b0_hbm, acc_sh.at[0], cnt0_sh,
                       idx2_v, rows2_v, ones_v, sems, wid, nw)
        _process_array(n1, x1_hbm, b1_hbm, acc_sh.at[1], cnt1_sh,
                       idx2_v, rows2_v, ones_v, sems, wid, nw)

        plsc.subcore_barrier()

        r0 = s * rows_per_tile
        for scale, cnt_sh in ((0, cnt0_sh), (1, cnt1_sh)):
            pltpu.sync_copy(acc_sh.at[scale, pl.ds(r0, rows_per_tile)],
                            acc_out.at[c, scale, pl.ds(r0, rows_per_tile)])

        @pl.when(s == 0)
        def _():
            pltpu.sync_copy(cnt0_sh,
                            cnt_out.at[pl.ds((c * 2 + 0) * _CNT_LEN,
                                             _CNT_LEN)])
            pltpu.sync_copy(cnt1_sh,
                            cnt_out.at[pl.ds((c * 2 + 1) * _CNT_LEN,
                                             _CNT_LEN)])

    return seg_kernel(x0, b0, x1, b1, zacc)


def _dense_body(acc_ref, cnt_ref, W1_ref, b1_ref, W2_ref, b2_ref, W3_ref,
                b3_ref, gamma_ref, beta_ref, out_ref):
    G = _NUM_GRAPHS
    s0 = acc_ref[0, 0, :G, :] + acc_ref[1, 0, :G, :]
    s1 = acc_ref[0, 1, :G, :] + acc_ref[1, 1, :G, :]
    c0 = cnt_ref[0, 0] + cnt_ref[1, 0]
    c1 = cnt_ref[0, 1] + cnt_ref[1, 1]
    g0 = s0 / jnp.maximum(c0, 1.0)
    g1 = s1 / jnp.maximum(c1, 1.0)

    h = jnp.dot(g0, W1_ref[:_HID, :], preferred_element_type=jnp.float32)
    h = h + jnp.dot(g1, W1_ref[_HID:, :], preferred_element_type=jnp.float32)
    h = jnp.maximum(h + b1_ref[...], 0.0)

    dv = W2_ref[:, 1:2] - W2_ref[:, 0:1]
    d = jnp.dot(h, dv, preferred_element_type=jnp.float32)
    d = d + (b2_ref[1:2] - b2_ref[0:1])
    w1 = 1.0 / (1.0 + jnp.exp(-d))
    w0 = 1.0 - w1

    ws = w0 * g0 + w1 * g1
    y = jnp.dot(ws, W3_ref[...], preferred_element_type=jnp.float32)
    y = jnp.maximum(y + b3_ref[...], 0.0)

    mean = jnp.mean(y, axis=-1, keepdims=True)
    var = jnp.mean((y - mean) ** 2, axis=-1, keepdims=True)
    out_ref[...] = ((y - mean) * lax.rsqrt(var + 1e-5) * gamma_ref[...]
                    + beta_ref[...])


def kernel(x0, batch0, x1, batch1, W1, b1, W2, b2, W3, b3, gamma, beta):
    zacc = jnp.zeros((_ACC_ROWS, _HID), jnp.float32)
    acc, cnt = _segment_sums(x0, batch0.astype(jnp.int32),
                             x1, batch1.astype(jnp.int32), zacc)
    nc = acc.shape[0]
    cnt = cnt.reshape(nc, 2, _CNT_LEN)[:, :, :_NUM_GRAPHS]
    cnt = cnt.reshape(nc, 2, _NUM_GRAPHS, 1)
    return pl.pallas_call(
        _dense_body,
        out_shape=jax.ShapeDtypeStruct((_NUM_GRAPHS, _HID), jnp.float32),
    )(acc, cnt, W1, b1, W2, b2, W3, b3, gamma, beta)


# speedup vs baseline: 1.1559x; 1.1559x over previous
import functools

import jax
import jax.numpy as jnp
from jax import lax
from jax.experimental import pallas as pl
from jax.experimental.pallas import tpu as pltpu
from jax.experimental.pallas import tpu_sc as plsc

_NUM_GRAPHS = 256
_HID = 128
_CHUNK = 128
_ACC_ROWS = 264
_DUMMY = 256
_CNT_LEN = 384


def _mask_tail_idx(idx_v, thresh):
    for k in range(_CHUNK // 16):
        vals = idx_v[pl.ds(k * 16, 16)]
        pos = lax.iota(jnp.int32, 16) + (k * 16)
        idx_v[pl.ds(k * 16, 16)] = jnp.where(pos < thresh, _DUMMY, vals)


def _process_array(n_rows, x_hbm, ids_hbm, acc_sh, cnt_sh, idx2_v, rows2_v,
                   ones_v, sems, wid, n_workers):
    n_chunks = (n_rows + _CHUNK - 1) // _CHUNK
    assert n_chunks >= 2 * n_workers
    tail_rem = n_rows - (n_chunks - 1) * _CHUNK
    tail_thresh = _CHUNK - tail_rem
    n_iters = (n_chunks - wid + n_workers - 1) // n_workers

    def rowbase(t):
        cid = wid + t * n_workers
        is_tail = cid == (n_chunks - 1)
        return jnp.where(is_tail, n_rows - _CHUNK, cid * _CHUNK), is_tail

    def start(t, b):
        rb, _ = rowbase(t)
        pltpu.async_copy(ids_hbm.at[pl.ds(rb, _CHUNK)], idx2_v.at[b],
                         sems.at[b])
        pltpu.async_copy(x_hbm.at[pl.ds(rb, _CHUNK)], rows2_v.at[b],
                         sems.at[b])

    def drain(b):
        pltpu.make_async_copy(ids_hbm.at[pl.ds(0, _CHUNK)], idx2_v.at[b],
                              sems.at[b]).wait()
        pltpu.make_async_copy(x_hbm.at[pl.ds(0, _CHUNK)], rows2_v.at[b],
                              sems.at[b]).wait()

    start(0, 0)
    start(1, 1)

    def pair_body(p, _):
        for b in range(2):
            t = 2 * p + b

            @pl.when(t < n_iters)
            def _():
                drain(b)
                _, is_tail = rowbase(t)
                if tail_thresh > 0:
                    @pl.when(is_tail)
                    def _():
                        _mask_tail_idx(idx2_v.at[b], tail_thresh)
                pltpu.sync_copy(rows2_v.at[b], acc_sh.at[idx2_v.at[b]],
                                add=True)
                pltpu.sync_copy(ones_v, cnt_sh.at[idx2_v.at[b]], add=True)

                @pl.when(t + 2 < n_iters)
                def _():
                    start(t + 2, b)
        return 0

    lax.fori_loop(0, (n_iters + 1) // 2, pair_body, 0)


def _segment_sums(x0, b0, x1, b1, zacc):
    info = plsc.get_sparse_core_info()
    nc, ns = info.num_cores, info.num_subcores
    nw = nc * ns
    mesh = plsc.VectorSubcoreMesh(core_axis_name="c", subcore_axis_name="s")
    n0 = x0.shape[0]
    n1 = x1.shape[0]
    rows_per_tile = _NUM_GRAPHS // ns

    @functools.partial(
        pl.kernel,
        out_type=[
            jax.ShapeDtypeStruct((nc, 2, _ACC_ROWS, _HID), jnp.float32),
            jax.ShapeDtypeStruct((nc * 2 * _CNT_LEN,), jnp.float32),
        ],
        mesh=mesh,
        scratch_types=[
            pltpu.VMEM((2, _CHUNK), jnp.int32),
            pltpu.VMEM((2, _CHUNK, _HID), jnp.float32),
            pltpu.VMEM((_CHUNK,), jnp.float32),
            pltpu.VMEM((_CNT_LEN,), jnp.float32),
            pltpu.VMEM_SHARED((2, _ACC_ROWS, _HID), jnp.float32),
            pltpu.VMEM_SHARED((_CNT_LEN,), jnp.float32),
            pltpu.VMEM_SHARED((_CNT_LEN,), jnp.float32),
            pltpu.SemaphoreType.DMA((2,)),
        ],
    )
    def seg_kernel(x0_hbm, b0_hbm, x1_hbm, b1_hbm, zacc_hbm,
                   acc_out, cnt_out, idx2_v, rows2_v, ones_v, cntz_v,
                   acc_sh, cnt0_sh, cnt1_sh, sems):
        c = lax.axis_index("c")
        s = lax.axis_index("s")
        wid = s * nc + c

        one = jnp.full((16,), 1.0, jnp.float32)
        for i in range(_CHUNK // 16):
            ones_v[pl.ds(i * 16, 16)] = one

        @pl.when(s == 0)
        def _():
            pltpu.sync_copy(zacc_hbm, acc_sh.at[0])
            pltpu.sync_copy(zacc_hbm, acc_sh.at[1])
            zero = jnp.zeros((16,), jnp.float32)
            for i in range(_CNT_LEN // 16):
                cntz_v[pl.ds(i * 16, 16)] = zero
            pltpu.sync_copy(cntz_v, cnt0_sh)
            pltpu.sync_copy(cntz_v, cnt1_sh)

        plsc.subcore_barrier()

        _process_array(n0, x0_hbm, b0_hbm, acc_sh.at[0], cnt0_sh,
                       idx2_v, rows2_v, ones_v, sems, wid, nw)
        _process_array(n1, x1_hbm, b1_hbm, acc_sh.at[1], cnt1_sh,
                       idx2_v, rows2_v, ones_v, sems, wid, nw)

        plsc.subcore_barrier()

        r0 = s * rows_per_tile
        for scale, cnt_sh in ((0, cnt0_sh), (1, cnt1_sh)):
            pltpu.sync_copy(acc_sh.at[scale, pl.ds(r0, rows_per_tile)],
                            acc_out.at[c, scale, pl.ds(r0, rows_per_tile)])

        @pl.when(s == 0)
        def _():
            pltpu.sync_copy(cnt0_sh,
                            cnt_out.at[pl.ds((c * 2 + 0) * _CNT_LEN,
                                             _CNT_LEN)])
            pltpu.sync_copy(cnt1_sh,
                            cnt_out.at[pl.ds((c * 2 + 1) * _CNT_LEN,
                                             _CNT_LEN)])

    return seg_kernel(x0, b0, x1, b1, zacc)


def _dense_body(acc_ref, cnt_ref, W1_ref, b1_ref, W2_ref, b2_ref, W3_ref,
                b3_ref, gamma_ref, beta_ref, out_ref):
    G = _NUM_GRAPHS
    s0 = acc_ref[0, 0, :G, :] + acc_ref[1, 0, :G, :]
    s1 = acc_ref[0, 1, :G, :] + acc_ref[1, 1, :G, :]
    c0 = jnp.reshape(cnt_ref[pl.ds(0 * _CNT_LEN, G)]
                     + cnt_ref[pl.ds(2 * _CNT_LEN, G)], (G, 1))
    c1 = jnp.reshape(cnt_ref[pl.ds(1 * _CNT_LEN, G)]
                     + cnt_ref[pl.ds(3 * _CNT_LEN, G)], (G, 1))
    g0 = s0 / jnp.maximum(c0, 1.0)
    g1 = s1 / jnp.maximum(c1, 1.0)

    h = jnp.dot(g0, W1_ref[:_HID, :], preferred_element_type=jnp.float32)
    h = h + jnp.dot(g1, W1_ref[_HID:, :], preferred_element_type=jnp.float32)
    h = jnp.maximum(h + b1_ref[...], 0.0)

    dv = W2_ref[:, 1:2] - W2_ref[:, 0:1]
    d = jnp.dot(h, dv, preferred_element_type=jnp.float32)
    d = d + (b2_ref[1:2] - b2_ref[0:1])
    w1 = 1.0 / (1.0 + jnp.exp(-d))
    w0 = 1.0 - w1

    ws = w0 * g0 + w1 * g1
    y = jnp.dot(ws, W3_ref[...], preferred_element_type=jnp.float32)
    y = jnp.maximum(y + b3_ref[...], 0.0)

    mean = jnp.mean(y, axis=-1, keepdims=True)
    var = jnp.mean((y - mean) ** 2, axis=-1, keepdims=True)
    out_ref[...] = ((y - mean) * lax.rsqrt(var + 1e-5) * gamma_ref[...]
                    + beta_ref[...])


def kernel(x0, batch0, x1, batch1, W1, b1, W2, b2, W3, b3, gamma, beta):
    zacc = jnp.zeros((_ACC_ROWS, _HID), jnp.float32)
    acc, cnt = _segment_sums(x0, batch0.astype(jnp.int32),
                             x1, batch1.astype(jnp.int32), zacc)
    return pl.pallas_call(
        _dense_body,
        out_shape=jax.ShapeDtypeStruct((_NUM_GRAPHS, _HID), jnp.float32),
    )(acc, cnt, W1, b1, W2, b2, W3, b3, gamma, beta)
